# Initial kernel scaffold; baseline (speedup 1.0000x reference)
import jax
import jax.numpy as jnp
from jax.experimental import pallas as pl


def kernel(features, edge_index, W1, b1, W2, b2, W3, b3, W4, b4):
    raise NotImplementedError("write your pallas kernel here")


# speedup vs baseline: 9.6935x; 9.6935x over previous
import jax
import jax.numpy as jnp
from jax import lax
from jax.experimental import pallas as pl
from jax.experimental.pallas import tpu as pltpu
from jax.experimental.pallas import tpu_sc as plsc

N = 10000
E = 320000
F_IN = 128
NC = 2
NS = 16
NW = NC * NS
CHUNK = 128
L = 16
SPAN = (E // NW) // CHUNK * CHUNK
NCHUNK = SPAN // CHUNK
NTAIL = (E - SPAN * NW) // CHUNK
NP = 10240
SLAB = NP // NS
R_TC = 1024
F32 = jnp.float32


def _sc_mesh():
    return plsc.VectorSubcoreMesh(
        core_axis_name="c", subcore_axis_name="s", num_cores=NC, num_subcores=NS)


def _load_spans(src_hbm, dst_hbm, srcall_v, dstall_v, w):
    pltpu.sync_copy(src_hbm.at[pl.ds(w * SPAN, SPAN)], srcall_v.at[pl.ds(0, SPAN)])
    pltpu.sync_copy(dst_hbm.at[pl.ds(w * SPAN, SPAN)], dstall_v.at[pl.ds(0, SPAN)])

    @pl.when(w < NTAIL)
    def _():
        t0 = SPAN * NW + w * CHUNK
        pltpu.sync_copy(src_hbm.at[pl.ds(t0, CHUNK)], srcall_v.at[pl.ds(SPAN, CHUNK)])
        pltpu.sync_copy(dst_hbm.at[pl.ds(t0, CHUNK)], dstall_v.at[pl.ds(SPAN, CHUNK)])


def _num_chunks(w):
    return jnp.where(w < NTAIL, NCHUNK + 1, NCHUNK)


def _deg_body(src_hbm, dst_hbm, zeros_hbm, deg_hbm,
              srcall_v, dstall_v, acc_src, acc_dst):
    c = lax.axis_index("c")
    s = lax.axis_index("s")
    w = c * NS + s
    _load_spans(src_hbm, dst_hbm, srcall_v, dstall_v, w)
    pltpu.sync_copy(zeros_hbm, acc_src)
    pltpu.sync_copy(zeros_hbm, acc_dst)

    def group(k, carry):
        sl = pl.ds(k * L, L)
        sv = srcall_v[sl]
        cnt_s, last_s = plsc.scan_count(sv)
        plsc.addupdate_scatter(acc_src, [sv], cnt_s.astype(F32), mask=last_s)
        dv = dstall_v[sl]
        cnt_d, last_d = plsc.scan_count(dv)
        plsc.addupdate_scatter(acc_dst, [dv], cnt_d.astype(F32), mask=last_d)
        return carry

    lax.fori_loop(0, _num_chunks(w) * (CHUNK // L), group, 0)
    pltpu.sync_copy(acc_src, deg_hbm.at[0, w, 0])
    pltpu.sync_copy(acc_dst, deg_hbm.at[1, w, 0])


def _agg_body(z_hbm, src_hbm, dst_hbm, zeros_hbm, out_hbm,
              srcall_v, dstall_v, sidx_v, didx_v, rows_v, accum_sh, sem):
    c = lax.axis_index("c")
    s = lax.axis_index("s")
    w = c * NS + s
    _load_spans(src_hbm, dst_hbm, srcall_v, dstall_v, w)
    for j in range(SLAB // CHUNK):
        pltpu.sync_copy(zeros_hbm, accum_sh.at[pl.ds(s * SLAB + j * CHUNK, CHUNK)])
    plsc.subcore_barrier()

    def body(k, carry):
        for j in range(CHUNK // L):
            sl = pl.ds(k * CHUNK + j * L, L)
            sidx_v[pl.ds(j * L, L)] = srcall_v[sl]
            didx_v[pl.ds(j * L, L)] = dstall_v[sl]
        pltpu.async_copy(z_hbm.at[sidx_v], rows_v, sem).wait()
        pltpu.sync_copy(rows_v, accum_sh.at[didx_v], add=True)
        return carry

    lax.fori_loop(0, _num_chunks(w), body, 0)
    plsc.subcore_barrier()
    pltpu.sync_copy(
        accum_sh.at[pl.ds(s * SLAB, SLAB)],
        out_hbm.at[pl.ds(c * NP + s * SLAB, SLAB)])


def _norm(d):
    return jnp.where(d > 0, lax.rsqrt(d), 0.0)


def _prep_body(x_ref, deg_ref, z_ref, no_ref, ni_ref):
    deg = deg_ref[...]
    no = _norm(jnp.sum(deg[:NW], axis=0))[:, None]
    ni = _norm(jnp.sum(deg[NW:], axis=0))[:, None]
    no_ref[...] = no
    ni_ref[...] = ni
    z_ref[...] = x_ref[...] * no


def _mid_body(aggp_ref, no_ref, ni_ref, w1_ref, b1_ref, w2_ref, z_ref):
    agg = (aggp_ref[0] + aggp_ref[1]) * ni_ref[...]
    h1 = jnp.maximum(
        jnp.dot(agg, w1_ref[...], preferred_element_type=F32) + b1_ref[...], 0.0)
    z_ref[...] = jnp.dot(h1, w2_ref[...], preferred_element_type=F32) * no_ref[...]


def _head_body(aggp_ref, ni_ref, b2_ref, w3_ref, b3_ref, w4_ref, b4_ref, o_ref):
    hidden = (aggp_ref[0] + aggp_ref[1]) * ni_ref[...] + b2_ref[...]
    h = jnp.maximum(
        jnp.dot(hidden, w3_ref[...], preferred_element_type=F32) + b3_ref[...], 0.0)
    o_ref[...] = jnp.dot(h, w4_ref[...], preferred_element_type=F32) + b4_ref[...]


def _whole(shape):
    return pl.BlockSpec(shape, lambda i: tuple(0 for _ in shape))


def kernel(features, edge_index, W1, b1, W2, b2, W3, b3, W4, b4):
    x = features.reshape(N, F_IN).astype(F32)
    ei = edge_index.astype(jnp.int32)
    src = ei[0]
    dst = ei[1]
    zeros_np = jnp.zeros((NP,), F32)
    zeros_slab = jnp.zeros((CHUNK, F_IN), F32)

    mesh = _sc_mesh()
    deg_call = pl.kernel(
        _deg_body,
        out_type=jax.ShapeDtypeStruct((2, NW, 1, NP), F32),
        mesh=mesh,
        compiler_params=pltpu.CompilerParams(needs_layout_passes=False),
        scratch_types=[
            pltpu.VMEM((SPAN + CHUNK,), jnp.int32),
            pltpu.VMEM((SPAN + CHUNK,), jnp.int32),
            pltpu.VMEM((NP,), F32),
            pltpu.VMEM((NP,), F32),
        ],
    )
    agg_call = pl.kernel(
        _agg_body,
        out_type=jax.ShapeDtypeStruct((NC * NP, F_IN), F32),
        mesh=mesh,
        scratch_types=[
            pltpu.VMEM((SPAN + CHUNK,), jnp.int32),
            pltpu.VMEM((SPAN + CHUNK,), jnp.int32),
            pltpu.VMEM((CHUNK,), jnp.int32),
            pltpu.VMEM((CHUNK,), jnp.int32),
            pltpu.VMEM((CHUNK, F_IN), F32),
            pltpu.VMEM_SHARED((NP, F_IN), F32),
            pltpu.SemaphoreType.DMA,
        ],
    )

    grid = ((N + R_TC - 1) // R_TC,)
    row_spec = lambda w: pl.BlockSpec((R_TC, w), lambda i: (i, 0))
    deg_spec = pl.BlockSpec((2 * NW, R_TC), lambda i: (0, i))
    aggp_spec = pl.BlockSpec((NC, R_TC, F_IN), lambda i: (0, i, 0))

    prep_call = pl.pallas_call(
        _prep_body, grid=grid,
        in_specs=[row_spec(F_IN), deg_spec],
        out_specs=(row_spec(F_IN), row_spec(1), row_spec(1)),
        out_shape=(jax.ShapeDtypeStruct((N, F_IN), F32),
                   jax.ShapeDtypeStruct((N, 1), F32),
                   jax.ShapeDtypeStruct((N, 1), F32)),
    )
    mid_call = pl.pallas_call(
        _mid_body, grid=grid,
        in_specs=[aggp_spec, row_spec(1), row_spec(1),
                  _whole((F_IN, 256)), _whole((1, 256)), _whole((256, F_IN))],
        out_specs=row_spec(F_IN),
        out_shape=jax.ShapeDtypeStruct((N, F_IN), F32),
    )
    head_call = pl.pallas_call(
        _head_body, grid=grid,
        in_specs=[aggp_spec, row_spec(1), _whole((1, F_IN)),
                  _whole((F_IN, 64)), _whole((1, 64)),
                  _whole((64, 1)), _whole((1, 1))],
        out_specs=row_spec(1),
        out_shape=jax.ShapeDtypeStruct((N, 1), F32),
    )

    deg = deg_call(src, dst, zeros_np).reshape(2 * NW, NP)
    z1, normo, normi = prep_call(x, deg)
    agg1 = agg_call(z1, src, dst, zeros_slab).reshape(NC, NP, F_IN)
    z2 = mid_call(agg1, normo, normi, W1, b1.reshape(1, -1), W2)
    agg2 = agg_call(z2, src, dst, zeros_slab).reshape(NC, NP, F_IN)
    out = head_call(agg2, normi, b2.reshape(1, -1), W3,
                    b3.reshape(1, -1), W4, b4.reshape(1, -1))
    return out


# speedup vs baseline: 10.7513x; 1.1091x over previous
import jax
import jax.numpy as jnp
from jax import lax
from jax.experimental import pallas as pl
from jax.experimental.pallas import tpu as pltpu
from jax.experimental.pallas import tpu_sc as plsc

N = 10000
E = 320000
F_IN = 128
NC = 2
NS = 16
NW = NC * NS
L = 16
SPAN = E // NW
ACH = 80
NCH = SPAN // ACH
NBUF = 2
NP = 10240
SLAB = NP // NS
ZROWS = 128
R_TC = 1024
F32 = jnp.float32


def _sc_mesh():
    return plsc.VectorSubcoreMesh(
        core_axis_name="c", subcore_axis_name="s", num_cores=NC, num_subcores=NS)


def _load_spans(src_hbm, dst_hbm, srcall_v, dstall_v, w):
    pltpu.sync_copy(src_hbm.at[pl.ds(w * SPAN, SPAN)], srcall_v)
    pltpu.sync_copy(dst_hbm.at[pl.ds(w * SPAN, SPAN)], dstall_v)


def _deg_body(src_hbm, dst_hbm, zeros_hbm, deg_hbm,
              srcall_v, dstall_v, acc_src, acc_dst):
    c = lax.axis_index("c")
    s = lax.axis_index("s")
    w = c * NS + s
    _load_spans(src_hbm, dst_hbm, srcall_v, dstall_v, w)
    pltpu.sync_copy(zeros_hbm, acc_src)
    pltpu.sync_copy(zeros_hbm, acc_dst)

    def group(k, carry):
        sl = pl.ds(k * L, L)
        sv = srcall_v[sl]
        cnt_s, last_s = plsc.scan_count(sv)
        plsc.addupdate_scatter(acc_src, [sv], cnt_s.astype(F32), mask=last_s)
        dv = dstall_v[sl]
        cnt_d, last_d = plsc.scan_count(dv)
        plsc.addupdate_scatter(acc_dst, [dv], cnt_d.astype(F32), mask=last_d)
        return carry

    lax.fori_loop(0, SPAN // L, group, 0)
    pltpu.sync_copy(acc_src, deg_hbm.at[0, w, 0])
    pltpu.sync_copy(acc_dst, deg_hbm.at[1, w, 0])


def _agg_body(z_hbm, src_hbm, dst_hbm, zeros_hbm, out_hbm,
              srcall_v, dstall_v, sidxs, didxs, rowss, gsems, ssems, accum_sh):
    c = lax.axis_index("c")
    s = lax.axis_index("s")
    w = c * NS + s
    _load_spans(src_hbm, dst_hbm, srcall_v, dstall_v, w)
    for j in range(SLAB // ZROWS):
        pltpu.sync_copy(zeros_hbm, accum_sh.at[pl.ds(s * SLAB + j * ZROWS, ZROWS)])
    plsc.subcore_barrier()

    def build(b, k):
        for j in range(ACH // L):
            sl = pl.ds(k * ACH + j * L, L)
            sidxs[b][pl.ds(j * L, L)] = srcall_v[sl]
            didxs[b][pl.ds(j * L, L)] = dstall_v[sl]

    def gather_start(b):
        pltpu.async_copy(z_hbm.at[sidxs[b]], rowss[b], gsems[b])

    def gather_wait(b):
        pltpu.make_async_copy(z_hbm.at[sidxs[b]], rowss[b], gsems[b]).wait()

    def scat_start(b):
        pltpu.async_copy(rowss[b], accum_sh.at[didxs[b]], ssems[b], add=True)

    def scat_wait(b):
        pltpu.make_async_copy(rowss[b], accum_sh.at[didxs[b]], ssems[b]).wait()

    for b in range(NBUF):
        build(b, b)
        gather_start(b)

    def body(i, carry):
        for b in range(NBUF):
            k = i * NBUF + b

            @pl.when(k < NCH)
            def _():
                gather_wait(b)
                scat_start(b)
        for b in range(NBUF):
            kn = i * NBUF + b + NBUF

            @pl.when(kn < NCH)
            def _():
                scat_wait(b)
                build(b, kn)
                gather_start(b)
        return carry

    lax.fori_loop(0, (NCH + NBUF - 1) // NBUF, body, 0)
    for b in range(NBUF):
        scat_wait(b)
    plsc.subcore_barrier()
    pltpu.sync_copy(
        accum_sh.at[pl.ds(s * SLAB, SLAB)],
        out_hbm.at[pl.ds(c * NP + s * SLAB, SLAB)])


def _norm(d):
    return jnp.where(d > 0, lax.rsqrt(d), 0.0)


def _prep_body(x_ref, deg_ref, z_ref, no_ref, ni_ref):
    deg = deg_ref[...]
    no = _norm(jnp.sum(deg[:NW], axis=0))[:, None]
    ni = _norm(jnp.sum(deg[NW:], axis=0))[:, None]
    no_ref[...] = no
    ni_ref[...] = ni
    z_ref[...] = x_ref[...] * no


def _mid_body(aggp_ref, no_ref, ni_ref, w1_ref, b1_ref, w2_ref, z_ref):
    agg = (aggp_ref[0] + aggp_ref[1]) * ni_ref[...]
    h1 = jnp.maximum(
        jnp.dot(agg, w1_ref[...], preferred_element_type=F32) + b1_ref[...], 0.0)
    z_ref[...] = jnp.dot(h1, w2_ref[...], preferred_element_type=F32) * no_ref[...]


def _head_body(aggp_ref, ni_ref, b2_ref, w3_ref, b3_ref, w4_ref, b4_ref, o_ref):
    hidden = (aggp_ref[0] + aggp_ref[1]) * ni_ref[...] + b2_ref[...]
    h = jnp.maximum(
        jnp.dot(hidden, w3_ref[...], preferred_element_type=F32) + b3_ref[...], 0.0)
    o_ref[...] = jnp.dot(h, w4_ref[...], preferred_element_type=F32) + b4_ref[...]


def _whole(shape):
    return pl.BlockSpec(shape, lambda i: tuple(0 for _ in shape))


def kernel(features, edge_index, W1, b1, W2, b2, W3, b3, W4, b4):
    x = features.reshape(N, F_IN).astype(F32)
    ei = edge_index.astype(jnp.int32)
    src = ei[0]
    dst = ei[1]
    zeros_np = jnp.zeros((NP,), F32)
    zeros_slab = jnp.zeros((ZROWS, F_IN), F32)

    mesh = _sc_mesh()
    deg_call = pl.kernel(
        _deg_body,
        out_type=jax.ShapeDtypeStruct((2, NW, 1, NP), F32),
        mesh=mesh,
        compiler_params=pltpu.CompilerParams(needs_layout_passes=False),
        scratch_types=[
            pltpu.VMEM((SPAN,), jnp.int32),
            pltpu.VMEM((SPAN,), jnp.int32),
            pltpu.VMEM((NP,), F32),
            pltpu.VMEM((NP,), F32),
        ],
    )
    agg_call = pl.kernel(
        _agg_body,
        out_type=jax.ShapeDtypeStruct((NC * NP, F_IN), F32),
        mesh=mesh,
        scratch_types=[
            pltpu.VMEM((SPAN,), jnp.int32),
            pltpu.VMEM((SPAN,), jnp.int32),
            [pltpu.VMEM((ACH,), jnp.int32) for _ in range(NBUF)],
            [pltpu.VMEM((ACH,), jnp.int32) for _ in range(NBUF)],
            [pltpu.VMEM((ACH, F_IN), F32) for _ in range(NBUF)],
            [pltpu.SemaphoreType.DMA for _ in range(NBUF)],
            [pltpu.SemaphoreType.DMA for _ in range(NBUF)],
            pltpu.VMEM_SHARED((NP, F_IN), F32),
        ],
    )

    grid = ((N + R_TC - 1) // R_TC,)
    row_spec = lambda w: pl.BlockSpec((R_TC, w), lambda i: (i, 0))
    deg_spec = pl.BlockSpec((2 * NW, R_TC), lambda i: (0, i))
    aggp_spec = pl.BlockSpec((NC, R_TC, F_IN), lambda i: (0, i, 0))

    prep_call = pl.pallas_call(
        _prep_body, grid=grid,
        in_specs=[row_spec(F_IN), deg_spec],
        out_specs=(row_spec(F_IN), row_spec(1), row_spec(1)),
        out_shape=(jax.ShapeDtypeStruct((N, F_IN), F32),
                   jax.ShapeDtypeStruct((N, 1), F32),
                   jax.ShapeDtypeStruct((N, 1), F32)),
    )
    mid_call = pl.pallas_call(
        _mid_body, grid=grid,
        in_specs=[aggp_spec, row_spec(1), row_spec(1),
                  _whole((F_IN, 256)), _whole((1, 256)), _whole((256, F_IN))],
        out_specs=row_spec(F_IN),
        out_shape=jax.ShapeDtypeStruct((N, F_IN), F32),
    )
    head_call = pl.pallas_call(
        _head_body, grid=grid,
        in_specs=[aggp_spec, row_spec(1), _whole((1, F_IN)),
                  _whole((F_IN, 64)), _whole((1, 64)),
                  _whole((64, 1)), _whole((1, 1))],
        out_specs=row_spec(1),
        out_shape=jax.ShapeDtypeStruct((N, 1), F32),
    )

    deg = deg_call(src, dst, zeros_np).reshape(2 * NW, NP)
    z1, normo, normi = prep_call(x, deg)
    agg1 = agg_call(z1, src, dst, zeros_slab).reshape(NC, NP, F_IN)
    z2 = mid_call(agg1, normo, normi, W1, b1.reshape(1, -1), W2)
    agg2 = agg_call(z2, src, dst, zeros_slab).reshape(NC, NP, F_IN)
    out = head_call(agg2, normi, b2.reshape(1, -1), W3,
                    b3.reshape(1, -1), W4, b4.reshape(1, -1))
    return out


# speedup vs baseline: 10.8369x; 1.0080x over previous
import jax
import jax.numpy as jnp
from jax import lax
from jax.experimental import pallas as pl
from jax.experimental.pallas import tpu as pltpu
from jax.experimental.pallas import tpu_sc as plsc

N = 10000
E = 320000
F_IN = 128
NC = 2
NS = 16
NW = NC * NS
L = 16
SPAN = E // NW
ACH = 80
NCH = SPAN // ACH
NBUF = 2
NP = 10240
SLAB = NP // NS
ZROWS = 128
R_TC = 1024
F32 = jnp.float32


def _sc_mesh():
    return plsc.VectorSubcoreMesh(
        core_axis_name="c", subcore_axis_name="s", num_cores=NC, num_subcores=NS)


def _deg_body(src_hbm, dst_hbm, zeros_hbm, deg_hbm,
              srcall_v, dstall_v, acc_src, acc_dst, sem0, sem1):
    c = lax.axis_index("c")
    s = lax.axis_index("s")
    w = c * NS + s
    pltpu.async_copy(src_hbm.at[pl.ds(w * SPAN, SPAN)], srcall_v, sem0)
    pltpu.async_copy(dst_hbm.at[pl.ds(w * SPAN, SPAN)], dstall_v, sem1)
    pltpu.sync_copy(zeros_hbm, acc_src)
    pltpu.sync_copy(zeros_hbm, acc_dst)
    pltpu.make_async_copy(src_hbm.at[pl.ds(w * SPAN, SPAN)], srcall_v, sem0).wait()
    pltpu.make_async_copy(dst_hbm.at[pl.ds(w * SPAN, SPAN)], dstall_v, sem1).wait()

    def group(k, carry):
        sl = pl.ds(k * L, L)
        sv = srcall_v[sl]
        cnt_s, last_s = plsc.scan_count(sv)
        plsc.addupdate_scatter(acc_src, [sv], cnt_s.astype(F32), mask=last_s)
        dv = dstall_v[sl]
        cnt_d, last_d = plsc.scan_count(dv)
        plsc.addupdate_scatter(acc_dst, [dv], cnt_d.astype(F32), mask=last_d)
        return carry

    lax.fori_loop(0, SPAN // L, group, 0)
    pltpu.sync_copy(acc_src, deg_hbm.at[0, w, 0])
    pltpu.sync_copy(acc_dst, deg_hbm.at[1, w, 0])


def _agg_body(z_hbm, src_hbm, dst_hbm, zeros_hbm, out_hbm,
              srcall_v, dstall_v, sidxs, didxs, rowss, gsems, ssems, accum_sh):
    c = lax.axis_index("c")
    s = lax.axis_index("s")
    w = c * NS + s
    pltpu.async_copy(src_hbm.at[pl.ds(w * SPAN, SPAN)], srcall_v, gsems[0])
    pltpu.async_copy(dst_hbm.at[pl.ds(w * SPAN, SPAN)], dstall_v, gsems[1])
    for j in range(SLAB // ZROWS):
        pltpu.async_copy(zeros_hbm, accum_sh.at[pl.ds(s * SLAB + j * ZROWS, ZROWS)],
                         ssems[0])
    pltpu.make_async_copy(src_hbm.at[pl.ds(w * SPAN, SPAN)], srcall_v, gsems[0]).wait()
    pltpu.make_async_copy(dst_hbm.at[pl.ds(w * SPAN, SPAN)], dstall_v, gsems[1]).wait()
    for j in range(SLAB // ZROWS):
        pltpu.make_async_copy(zeros_hbm,
                              accum_sh.at[pl.ds(s * SLAB + j * ZROWS, ZROWS)],
                              ssems[0]).wait()
    plsc.subcore_barrier()

    def build(b, k):
        for j in range(ACH // L):
            sl = pl.ds(k * ACH + j * L, L)
            sidxs[b][pl.ds(j * L, L)] = srcall_v[sl]
            didxs[b][pl.ds(j * L, L)] = dstall_v[sl]

    def gather_start(b):
        pltpu.async_copy(z_hbm.at[sidxs[b]], rowss[b], gsems[b])

    def gather_wait(b):
        pltpu.make_async_copy(z_hbm.at[sidxs[b]], rowss[b], gsems[b]).wait()

    def scat_start(b):
        pltpu.async_copy(rowss[b], accum_sh.at[didxs[b]], ssems[b], add=True)

    def scat_wait(b):
        pltpu.make_async_copy(rowss[b], accum_sh.at[didxs[b]], ssems[b]).wait()

    for b in range(NBUF):
        build(b, b)
        gather_start(b)

    def body(i, carry):
        for b in range(NBUF):
            k = i * NBUF + b

            @pl.when(k < NCH)
            def _():
                gather_wait(b)
                scat_start(b)
        for b in range(NBUF):
            kn = i * NBUF + b + NBUF

            @pl.when(kn < NCH)
            def _():
                scat_wait(b)
                build(b, kn)
                gather_start(b)
        return carry

    lax.fori_loop(0, (NCH + NBUF - 1) // NBUF, body, 0)
    for b in range(NBUF):
        scat_wait(b)
    plsc.subcore_barrier()
    pltpu.sync_copy(
        accum_sh.at[pl.ds(s * SLAB, SLAB)],
        out_hbm.at[pl.ds(c * NP + s * SLAB, SLAB)])


def _norm(d):
    return jnp.where(d > 0, lax.rsqrt(d), 0.0)


def _prep_body(x_ref, deg_ref, z_ref, no_ref, ni_ref):
    deg = deg_ref[...]
    no = _norm(jnp.sum(deg[:NW], axis=0))[:, None]
    ni = _norm(jnp.sum(deg[NW:], axis=0))[:, None]
    no_ref[...] = no
    ni_ref[...] = ni
    z_ref[...] = x_ref[...] * no


def _mid_body(aggp_ref, no_ref, ni_ref, w1_ref, b1_ref, w2_ref, z_ref):
    agg = (aggp_ref[0] + aggp_ref[1]) * ni_ref[...]
    h1 = jnp.maximum(
        jnp.dot(agg, w1_ref[...], preferred_element_type=F32) + b1_ref[...], 0.0)
    z_ref[...] = jnp.dot(h1, w2_ref[...], preferred_element_type=F32) * no_ref[...]


def _head_body(aggp_ref, ni_ref, b2_ref, w3_ref, b3_ref, w4_ref, b4_ref, o_ref):
    hidden = (aggp_ref[0] + aggp_ref[1]) * ni_ref[...] + b2_ref[...]
    h = jnp.maximum(
        jnp.dot(hidden, w3_ref[...], preferred_element_type=F32) + b3_ref[...], 0.0)
    o_ref[...] = jnp.dot(h, w4_ref[...], preferred_element_type=F32) + b4_ref[...]


def _whole(shape):
    return pl.BlockSpec(shape, lambda i: tuple(0 for _ in shape))


def kernel(features, edge_index, W1, b1, W2, b2, W3, b3, W4, b4):
    x = features.reshape(N, F_IN).astype(F32)
    ei = edge_index.astype(jnp.int32)
    src = ei[0]
    dst = ei[1]
    zeros_np = jnp.zeros((NP,), F32)
    zeros_slab = jnp.zeros((ZROWS, F_IN), F32)

    mesh = _sc_mesh()
    deg_call = pl.kernel(
        _deg_body,
        out_type=jax.ShapeDtypeStruct((2, NW, 1, NP), F32),
        mesh=mesh,
        compiler_params=pltpu.CompilerParams(needs_layout_passes=False),
        scratch_types=[
            pltpu.VMEM((SPAN,), jnp.int32),
            pltpu.VMEM((SPAN,), jnp.int32),
            pltpu.VMEM((NP,), F32),
            pltpu.VMEM((NP,), F32),
            pltpu.SemaphoreType.DMA,
            pltpu.SemaphoreType.DMA,
        ],
    )
    agg_call = pl.kernel(
        _agg_body,
        out_type=jax.ShapeDtypeStruct((NC * NP, F_IN), F32),
        mesh=mesh,
        scratch_types=[
            pltpu.VMEM((SPAN,), jnp.int32),
            pltpu.VMEM((SPAN,), jnp.int32),
            [pltpu.VMEM((ACH,), jnp.int32) for _ in range(NBUF)],
            [pltpu.VMEM((ACH,), jnp.int32) for _ in range(NBUF)],
            [pltpu.VMEM((ACH, F_IN), F32) for _ in range(NBUF)],
            [pltpu.SemaphoreType.DMA for _ in range(NBUF)],
            [pltpu.SemaphoreType.DMA for _ in range(NBUF)],
            pltpu.VMEM_SHARED((NP, F_IN), F32),
        ],
    )

    grid = ((N + R_TC - 1) // R_TC,)
    row_spec = lambda w: pl.BlockSpec((R_TC, w), lambda i: (i, 0))
    deg_spec = pl.BlockSpec((2 * NW, R_TC), lambda i: (0, i))
    aggp_spec = pl.BlockSpec((NC, R_TC, F_IN), lambda i: (0, i, 0))

    prep_call = pl.pallas_call(
        _prep_body, grid=grid,
        in_specs=[row_spec(F_IN), deg_spec],
        out_specs=(row_spec(F_IN), row_spec(1), row_spec(1)),
        out_shape=(jax.ShapeDtypeStruct((N, F_IN), F32),
                   jax.ShapeDtypeStruct((N, 1), F32),
                   jax.ShapeDtypeStruct((N, 1), F32)),
    )
    mid_call = pl.pallas_call(
        _mid_body, grid=grid,
        in_specs=[aggp_spec, row_spec(1), row_spec(1),
                  _whole((F_IN, 256)), _whole((1, 256)), _whole((256, F_IN))],
        out_specs=row_spec(F_IN),
        out_shape=jax.ShapeDtypeStruct((N, F_IN), F32),
    )
    head_call = pl.pallas_call(
        _head_body, grid=grid,
        in_specs=[aggp_spec, row_spec(1), _whole((1, F_IN)),
                  _whole((F_IN, 64)), _whole((1, 64)),
                  _whole((64, 1)), _whole((1, 1))],
        out_specs=row_spec(1),
        out_shape=jax.ShapeDtypeStruct((N, 1), F32),
    )

    deg = deg_call(src, dst, zeros_np).reshape(2 * NW, NP)
    z1, normo, normi = prep_call(x, deg)
    agg1 = agg_call(z1, src, dst, zeros_slab).reshape(NC, NP, F_IN)
    z2 = mid_call(agg1, normo, normi, W1, b1.reshape(1, -1), W2)
    agg2 = agg_call(z2, src, dst, zeros_slab).reshape(NC, NP, F_IN)
    out = head_call(agg2, normi, b2.reshape(1, -1), W3,
                    b3.reshape(1, -1), W4, b4.reshape(1, -1))
    return out


# speedup vs baseline: 11.0318x; 1.0180x over previous
import jax
import jax.numpy as jnp
from jax import lax
from jax.experimental import pallas as pl
from jax.experimental.pallas import tpu as pltpu
from jax.experimental.pallas import tpu_sc as plsc

N = 10000
E = 320000
F_IN = 128
NC = 2
NS = 16
NW = NC * NS
L = 16
SPAN = E // NW
ACH = 80
NCH = SPAN // ACH
NBUF = 2
NP = 10240
SLAB = NP // NS
ZROWS = 128
R_TC = 2048
F32 = jnp.float32


def _sc_mesh():
    return plsc.VectorSubcoreMesh(
        core_axis_name="c", subcore_axis_name="s", num_cores=NC, num_subcores=NS)


def _deg_body(src_hbm, dst_hbm, zeros_hbm, deg_hbm,
              srcall_v, dstall_v, acc_src, acc_dst, sem0, sem1):
    c = lax.axis_index("c")
    s = lax.axis_index("s")
    w = c * NS + s
    pltpu.async_copy(src_hbm.at[pl.ds(w * SPAN, SPAN)], srcall_v, sem0)
    pltpu.async_copy(dst_hbm.at[pl.ds(w * SPAN, SPAN)], dstall_v, sem1)
    pltpu.sync_copy(zeros_hbm, acc_src)
    pltpu.sync_copy(zeros_hbm, acc_dst)
    pltpu.make_async_copy(src_hbm.at[pl.ds(w * SPAN, SPAN)], srcall_v, sem0).wait()
    pltpu.make_async_copy(dst_hbm.at[pl.ds(w * SPAN, SPAN)], dstall_v, sem1).wait()

    def group(k, carry):
        sl = pl.ds(k * L, L)
        sv = srcall_v[sl]
        cnt_s, last_s = plsc.scan_count(sv)
        plsc.addupdate_scatter(acc_src, [sv], cnt_s.astype(F32), mask=last_s)
        dv = dstall_v[sl]
        cnt_d, last_d = plsc.scan_count(dv)
        plsc.addupdate_scatter(acc_dst, [dv], cnt_d.astype(F32), mask=last_d)
        return carry

    lax.fori_loop(0, SPAN // L, group, 0)
    pltpu.sync_copy(acc_src, deg_hbm.at[0, w, 0])
    pltpu.sync_copy(acc_dst, deg_hbm.at[1, w, 0])


def _agg_body(z_hbm, src_hbm, dst_hbm, zeros_hbm, out_hbm,
              srcall_v, dstall_v, sidxs, didxs, rowss, gsems, ssems, accum_sh):
    c = lax.axis_index("c")
    s = lax.axis_index("s")
    w = c * NS + s
    pltpu.async_copy(src_hbm.at[pl.ds(w * SPAN, SPAN)], srcall_v, gsems[0])
    pltpu.async_copy(dst_hbm.at[pl.ds(w * SPAN, SPAN)], dstall_v, gsems[1])
    for j in range(SLAB // ZROWS):
        pltpu.async_copy(zeros_hbm, accum_sh.at[pl.ds(s * SLAB + j * ZROWS, ZROWS)],
                         ssems[0])
    pltpu.make_async_copy(src_hbm.at[pl.ds(w * SPAN, SPAN)], srcall_v, gsems[0]).wait()
    pltpu.make_async_copy(dst_hbm.at[pl.ds(w * SPAN, SPAN)], dstall_v, gsems[1]).wait()
    for j in range(SLAB // ZROWS):
        pltpu.make_async_copy(zeros_hbm,
                              accum_sh.at[pl.ds(s * SLAB + j * ZROWS, ZROWS)],
                              ssems[0]).wait()
    plsc.subcore_barrier()

    def build(b, k):
        for j in range(ACH // L):
            sl = pl.ds(k * ACH + j * L, L)
            sidxs[b][pl.ds(j * L, L)] = srcall_v[sl]
            didxs[b][pl.ds(j * L, L)] = dstall_v[sl]

    def gather_start(b):
        pltpu.async_copy(z_hbm.at[sidxs[b]], rowss[b], gsems[b])

    def gather_wait(b):
        pltpu.make_async_copy(z_hbm.at[sidxs[b]], rowss[b], gsems[b]).wait()

    def scat_start(b):
        pltpu.async_copy(rowss[b], accum_sh.at[didxs[b]], ssems[b], add=True)

    def scat_wait(b):
        pltpu.make_async_copy(rowss[b], accum_sh.at[didxs[b]], ssems[b]).wait()

    for b in range(NBUF):
        build(b, b)
        gather_start(b)

    def body(i, carry):
        for b in range(NBUF):
            k = i * NBUF + b

            @pl.when(k < NCH)
            def _():
                gather_wait(b)
                scat_start(b)
        for b in range(NBUF):
            kn = i * NBUF + b + NBUF

            @pl.when(kn < NCH)
            def _():
                scat_wait(b)
                build(b, kn)
                gather_start(b)
        return carry

    lax.fori_loop(0, (NCH + NBUF - 1) // NBUF, body, 0)
    for b in range(NBUF):
        scat_wait(b)
    plsc.subcore_barrier()
    pltpu.sync_copy(
        accum_sh.at[pl.ds(s * SLAB, SLAB)],
        out_hbm.at[pl.ds(c * NP + s * SLAB, SLAB)])


def _norm(d):
    return jnp.where(d > 0, lax.rsqrt(d), 0.0)


def _prep_body(x_ref, deg_ref, z_ref, no_ref, ni_ref):
    deg = deg_ref[...]
    no = _norm(jnp.sum(deg[:NW], axis=0))[:, None]
    ni = _norm(jnp.sum(deg[NW:], axis=0))[:, None]
    no_ref[...] = no
    ni_ref[...] = ni
    z_ref[...] = x_ref[...] * no


def _mid_body(aggp_ref, no_ref, ni_ref, w1_ref, b1_ref, w2_ref, z_ref):
    agg = (aggp_ref[0] + aggp_ref[1]) * ni_ref[...]
    h1 = jnp.maximum(
        jnp.dot(agg, w1_ref[...], preferred_element_type=F32) + b1_ref[...], 0.0)
    z_ref[...] = jnp.dot(h1, w2_ref[...], preferred_element_type=F32) * no_ref[...]


def _head_body(aggp_ref, ni_ref, b2_ref, w3_ref, b3_ref, w4_ref, b4_ref, o_ref):
    hidden = (aggp_ref[0] + aggp_ref[1]) * ni_ref[...] + b2_ref[...]
    h = jnp.maximum(
        jnp.dot(hidden, w3_ref[...], preferred_element_type=F32) + b3_ref[...], 0.0)
    o_ref[...] = jnp.dot(h, w4_ref[...], preferred_element_type=F32) + b4_ref[...]


def _whole(shape):
    return pl.BlockSpec(shape, lambda i: tuple(0 for _ in shape))


def kernel(features, edge_index, W1, b1, W2, b2, W3, b3, W4, b4):
    x = features.reshape(N, F_IN).astype(F32)
    ei = edge_index.astype(jnp.int32)
    src = ei[0]
    dst = ei[1]
    zeros_np = jnp.zeros((NP,), F32)
    zeros_slab = jnp.zeros((ZROWS, F_IN), F32)

    mesh = _sc_mesh()
    deg_call = pl.kernel(
        _deg_body,
        out_type=jax.ShapeDtypeStruct((2, NW, 1, NP), F32),
        mesh=mesh,
        compiler_params=pltpu.CompilerParams(needs_layout_passes=False),
        scratch_types=[
            pltpu.VMEM((SPAN,), jnp.int32),
            pltpu.VMEM((SPAN,), jnp.int32),
            pltpu.VMEM((NP,), F32),
            pltpu.VMEM((NP,), F32),
            pltpu.SemaphoreType.DMA,
            pltpu.SemaphoreType.DMA,
        ],
    )
    agg_call = pl.kernel(
        _agg_body,
        out_type=jax.ShapeDtypeStruct((NC * NP, F_IN), F32),
        mesh=mesh,
        scratch_types=[
            pltpu.VMEM((SPAN,), jnp.int32),
            pltpu.VMEM((SPAN,), jnp.int32),
            [pltpu.VMEM((ACH,), jnp.int32) for _ in range(NBUF)],
            [pltpu.VMEM((ACH,), jnp.int32) for _ in range(NBUF)],
            [pltpu.VMEM((ACH, F_IN), F32) for _ in range(NBUF)],
            [pltpu.SemaphoreType.DMA for _ in range(NBUF)],
            [pltpu.SemaphoreType.DMA for _ in range(NBUF)],
            pltpu.VMEM_SHARED((NP, F_IN), F32),
        ],
    )

    grid = ((N + R_TC - 1) // R_TC,)
    row_spec = lambda w: pl.BlockSpec((R_TC, w), lambda i: (i, 0))
    deg_spec = pl.BlockSpec((2 * NW, R_TC), lambda i: (0, i))
    aggp_spec = pl.BlockSpec((NC, R_TC, F_IN), lambda i: (0, i, 0))

    prep_call = pl.pallas_call(
        _prep_body, grid=grid,
        in_specs=[row_spec(F_IN), deg_spec],
        out_specs=(row_spec(F_IN), row_spec(1), row_spec(1)),
        out_shape=(jax.ShapeDtypeStruct((N, F_IN), F32),
                   jax.ShapeDtypeStruct((N, 1), F32),
                   jax.ShapeDtypeStruct((N, 1), F32)),
    )
    mid_call = pl.pallas_call(
        _mid_body, grid=grid,
        in_specs=[aggp_spec, row_spec(1), row_spec(1),
                  _whole((F_IN, 256)), _whole((1, 256)), _whole((256, F_IN))],
        out_specs=row_spec(F_IN),
        out_shape=jax.ShapeDtypeStruct((N, F_IN), F32),
    )
    head_call = pl.pallas_call(
        _head_body, grid=grid,
        in_specs=[aggp_spec, row_spec(1), _whole((1, F_IN)),
                  _whole((F_IN, 64)), _whole((1, 64)),
                  _whole((64, 1)), _whole((1, 1))],
        out_specs=row_spec(1),
        out_shape=jax.ShapeDtypeStruct((N, 1), F32),
    )

    deg = deg_call(src, dst, zeros_np).reshape(2 * NW, NP)
    z1, normo, normi = prep_call(x, deg)
    agg1 = agg_call(z1, src, dst, zeros_slab).reshape(NC, NP, F_IN)
    z2 = mid_call(agg1, normo, normi, W1, b1.reshape(1, -1), W2)
    agg2 = agg_call(z2, src, dst, zeros_slab).reshape(NC, NP, F_IN)
    out = head_call(agg2, normi, b2.reshape(1, -1), W3,
                    b3.reshape(1, -1), W4, b4.reshape(1, -1))
    return out
